# baseline (speedup 1.0000x reference)
import functools

import jax
import jax.numpy as jnp
from jax.experimental import pallas as pl
from jax.experimental.pallas import tpu as pltpu


def _chamfer_kernel(pt_ref, g_ref, pth_ref, gh_ref,
                    out_ref, colmin_ref, colval_ref, facc_ref,
                    *, n_row_tiles):
    i = pl.program_id(1)

    p = pt_ref[0]
    g = g_ref[0]
    px = p[:, 0:1]
    py = p[:, 1:2]
    pz = p[:, 2:3]
    gx = g[0:1, :]
    gy = g[1:2, :]
    gz = g[2:3, :]

    p2 = px * px + py * py + pz * pz
    g2 = gx * gx + gy * gy + gz * gz
    s = p2 + g2

    ph = pth_ref[0]
    gh = gh_ref[0]

    cross_sel = jnp.dot(ph, gh, preferred_element_type=jnp.float32)
    cross_ex = jnp.dot(p, g, preferred_element_type=jnp.float32,
                       precision=jax.lax.Precision.HIGHEST)

    d_sel = s - 2.0 * cross_sel
    d_ex = s - 2.0 * cross_ex

    inf = jnp.float32(jnp.inf)

    rowmin = jnp.min(d_sel, axis=1, keepdims=True)
    rowval = jnp.min(jnp.where(d_sel == rowmin, d_ex, inf),
                     axis=1, keepdims=True)
    fwd = jnp.sum(jnp.sqrt(jnp.maximum(rowval, 0.0) + 1e-8))

    tile_colmin = jnp.min(d_sel, axis=0, keepdims=True)
    tile_colval = jnp.min(jnp.where(d_sel == tile_colmin, d_ex, inf),
                          axis=0, keepdims=True)

    @pl.when(i == 0)
    def _init_batch():
        facc_ref[0, 0] = 0.0
        colmin_ref[...] = tile_colmin
        colval_ref[...] = tile_colval

    facc_ref[0, 0] += fwd

    @pl.when(i > 0)
    def _update_col():
        better = tile_colmin < colmin_ref[...]
        colmin_ref[...] = jnp.where(better, tile_colmin, colmin_ref[...])
        colval_ref[...] = jnp.where(better, tile_colval, colval_ref[...])

    @pl.when(i == n_row_tiles - 1)
    def _finish_batch():
        bwd = jnp.sum(jnp.sqrt(jnp.maximum(colval_ref[...], 0.0) + 1e-8))
        lane = jax.lax.broadcasted_iota(jnp.int32, (1, 128), 1)
        vec = jnp.where(lane == 0, facc_ref[0, 0],
                        jnp.where(lane == 1, bwd, 0.0))
        out_ref[...] = vec[None]


def kernel(predict_pc, gt_pc):
    B, C, M = predict_pc.shape
    N = gt_pc.shape[2]
    TM = 512
    n_row_tiles = M // TM

    pt = jnp.transpose(predict_pc, (0, 2, 1))
    pth = pt.astype(jnp.bfloat16)
    gh = gt_pc.astype(jnp.bfloat16)

    out = pl.pallas_call(
        functools.partial(_chamfer_kernel, n_row_tiles=n_row_tiles),
        grid=(B, n_row_tiles),
        in_specs=[
            pl.BlockSpec((1, TM, C), lambda b, i: (b, i, 0)),
            pl.BlockSpec((1, C, N), lambda b, i: (b, 0, 0)),
            pl.BlockSpec((1, TM, C), lambda b, i: (b, i, 0)),
            pl.BlockSpec((1, C, N), lambda b, i: (b, 0, 0)),
        ],
        out_specs=pl.BlockSpec((1, 1, 128), lambda b, i: (b, 0, 0)),
        out_shape=jax.ShapeDtypeStruct((B, 1, 128), jnp.float32),
        scratch_shapes=[
            pltpu.VMEM((1, N), jnp.float32),
            pltpu.VMEM((1, N), jnp.float32),
            pltpu.SMEM((1, 1), jnp.float32),
        ],
        compiler_params=pltpu.CompilerParams(
            dimension_semantics=("parallel", "arbitrary")),
    )(pt, gt_pc, pth, gh)
    fwd_total = jnp.sum(out[:, 0, 0])
    bwd_total = jnp.sum(out[:, 0, 1])
    return fwd_total / (B * M) + bwd_total / (B * N)


# speedup vs baseline: 1.6105x; 1.6105x over previous
import functools

import jax
import jax.numpy as jnp
from jax.experimental import pallas as pl
from jax.experimental.pallas import tpu as pltpu


def _chamfer_kernel(pt_ref, g_ref, g2x_ref, pth_ref, gh_ref,
                    out_ref, colmin_ref, colval_ref, facc_ref,
                    *, n_row_tiles):
    i = pl.program_id(1)

    p = pt_ref[0]
    g = g_ref[0]
    px = p[:, 0:1]
    py = p[:, 1:2]
    pz = p[:, 2:3]
    gx = g[0:1, :]
    gy = g[1:2, :]
    gz = g[2:3, :]

    p2 = px * px + py * py + pz * pz
    g2 = gx * gx + gy * gy + gz * gz
    s = p2 + g2

    ph = pth_ref[0]
    gh = gh_ref[0]

    cross2_sel = jnp.dot(ph, gh, preferred_element_type=jnp.float32)
    g2x = g2x_ref[0]
    cross2_ex = (px * g2x[0:1, :] + py * g2x[1:2, :] + pz * g2x[2:3, :])

    d_sel = s - cross2_sel
    d_ex = s - cross2_ex

    inf = jnp.float32(jnp.inf)

    rowmin = jnp.min(d_sel, axis=1, keepdims=True)
    rowval = jnp.min(jnp.where(d_sel == rowmin, d_ex, inf),
                     axis=1, keepdims=True)
    fwd = jnp.sum(jnp.sqrt(jnp.maximum(rowval, 0.0) + 1e-8))

    tile_colmin = jnp.min(d_sel, axis=0, keepdims=True)
    tile_colval = jnp.min(jnp.where(d_sel == tile_colmin, d_ex, inf),
                          axis=0, keepdims=True)

    @pl.when(i == 0)
    def _init_batch():
        facc_ref[0, 0] = 0.0
        colmin_ref[...] = tile_colmin
        colval_ref[...] = tile_colval

    facc_ref[0, 0] += fwd

    @pl.when(i > 0)
    def _update_col():
        better = tile_colmin < colmin_ref[...]
        colmin_ref[...] = jnp.where(better, tile_colmin, colmin_ref[...])
        colval_ref[...] = jnp.where(better, tile_colval, colval_ref[...])

    @pl.when(i == n_row_tiles - 1)
    def _finish_batch():
        bwd = jnp.sum(jnp.sqrt(jnp.maximum(colval_ref[...], 0.0) + 1e-8))
        lane = jax.lax.broadcasted_iota(jnp.int32, (1, 128), 1)
        vec = jnp.where(lane == 0, facc_ref[0, 0],
                        jnp.where(lane == 1, bwd, 0.0))
        out_ref[...] = vec[None]


def kernel(predict_pc, gt_pc):
    B, C, M = predict_pc.shape
    N = gt_pc.shape[2]
    TM = 512
    n_row_tiles = M // TM

    pt = jnp.transpose(predict_pc, (0, 2, 1))
    gt2x = gt_pc * 2.0
    pth = pt.astype(jnp.bfloat16)
    gh2x = gt2x.astype(jnp.bfloat16)

    out = pl.pallas_call(
        functools.partial(_chamfer_kernel, n_row_tiles=n_row_tiles),
        grid=(B, n_row_tiles),
        in_specs=[
            pl.BlockSpec((1, TM, C), lambda b, i: (b, i, 0)),
            pl.BlockSpec((1, C, N), lambda b, i: (b, 0, 0)),
            pl.BlockSpec((1, C, N), lambda b, i: (b, 0, 0)),
            pl.BlockSpec((1, TM, C), lambda b, i: (b, i, 0)),
            pl.BlockSpec((1, C, N), lambda b, i: (b, 0, 0)),
        ],
        out_specs=pl.BlockSpec((1, 1, 128), lambda b, i: (b, 0, 0)),
        out_shape=jax.ShapeDtypeStruct((B, 1, 128), jnp.float32),
        scratch_shapes=[
            pltpu.VMEM((1, N), jnp.float32),
            pltpu.VMEM((1, N), jnp.float32),
            pltpu.SMEM((1, 1), jnp.float32),
        ],
        compiler_params=pltpu.CompilerParams(
            dimension_semantics=("parallel", "arbitrary")),
    )(pt, gt_pc, gt2x, pth, gh2x)
    fwd_total = jnp.sum(out[:, 0, 0])
    bwd_total = jnp.sum(out[:, 0, 1])
    return fwd_total / (B * M) + bwd_total / (B * N)
